# baseline (speedup 1.0000x reference)
import functools

import jax
import jax.numpy as jnp
from jax import lax
from jax.experimental import pallas as pl
from jax.experimental.pallas import tpu as pltpu
from jax.experimental.pallas import tpu_sc as plsc

B, C_IN, N, K = 4, 9, 4096, 20
TN = 256
NT = N // TN
EPS = 1e-5
CNT_E = float(B * N * K)
CNT_P = float(B * N)
_INTERPRET = False


def _dotb(x, w):
    return lax.dot_general(x.astype(jnp.bfloat16), w.astype(jnp.bfloat16),
                           (((1,), (1,)), ((), ())),
                           preferred_element_type=jnp.float32)


def _affine(s, g, b, cnt):
    m = s[0:1, :] / cnt
    v = s[1:2, :] / cnt - m * m
    al = g * lax.rsqrt(v + EPS)
    be = b - al * m
    return al, be


def _xnext_body(mx_ref, mn_ref, st_ref, g_ref, b_ref, x_ref):
    al, be = _affine(st_ref[...], g_ref[...], b_ref[...], CNT_E)
    x_ref[0] = jnp.maximum(jnp.where(al >= 0, mx_ref[0], mn_ref[0]) * al + be,
                           0.0)


def _edge_h1(xg_ref, x_ref, w0_ref):
    cp = x_ref.shape[2]
    xc = x_ref[0]
    wc = w0_ref[:, :cp]
    wd = w0_ref[:, cp:]
    hc = _dotb(xc, wc)
    xg2 = xg_ref[0].reshape(K * TN, cp)
    xcr = jnp.broadcast_to(xc[None], (K, TN, cp)).reshape(K * TN, cp)
    hcr = jnp.broadcast_to(hc[None], (K, TN, 64)).reshape(K * TN, 64)
    return hcr + _dotb(xg2 - xcr, wd)


def _stats_body(xg_ref, x_ref, w0_ref, st_ref):
    i = pl.program_id(0)
    j = pl.program_id(1)
    h = _edge_h1(xg_ref, x_ref, w0_ref)

    @pl.when(jnp.logical_and(i == 0, j == 0))
    def _():
        st_ref[...] = jnp.zeros((2, 64), jnp.float32)

    st_ref[0:1, :] += jnp.sum(h, axis=0, keepdims=True)
    st_ref[1:2, :] += jnp.sum(h * h, axis=0, keepdims=True)


def _edge_body(xg_ref, x_ref, w0_ref, st1_ref, g_ref, b_ref, w2_ref,
               mx_ref, mn_ref, st2_ref):
    i = pl.program_id(0)
    j = pl.program_id(1)
    al, be = _affine(st1_ref[...], g_ref[...], b_ref[...], CNT_E)
    h = _edge_h1(xg_ref, x_ref, w0_ref)
    a = jnp.maximum(h * al + be, 0.0)
    h2 = _dotb(a, w2_ref[...]).reshape(K, TN, 64)
    mx_ref[0] = jnp.max(h2, axis=0)
    mn_ref[0] = jnp.min(h2, axis=0)

    @pl.when(jnp.logical_and(i == 0, j == 0))
    def _():
        st2_ref[...] = jnp.zeros((2, 64), jnp.float32)

    h2f = h2.reshape(K * TN, 64)
    st2_ref[0:1, :] += jnp.sum(h2f, axis=0, keepdims=True)
    st2_ref[1:2, :] += jnp.sum(h2f * h2f, axis=0, keepdims=True)


def _local_body(mx_ref, mn_ref, st_ref, g_ref, b_ref, x1_ref, x2_ref, lw_ref,
                x3_ref, gmm_ref, lst_ref):
    i = pl.program_id(0)
    j = pl.program_id(1)
    al, be = _affine(st_ref[...], g_ref[...], b_ref[...], CNT_E)
    x3 = jnp.maximum(jnp.where(al >= 0, mx_ref[0], mn_ref[0]) * al + be, 0.0)
    x3_ref[0] = x3
    loc = (_dotb(x1_ref[0], lw_ref[:, :64])
           + _dotb(x2_ref[0], lw_ref[:, 64:128])
           + _dotb(x3, lw_ref[:, 128:]))
    rmax = jnp.max(loc, axis=0, keepdims=True)
    rmin = jnp.min(loc, axis=0, keepdims=True)

    @pl.when(j == 0)
    def _():
        gmm_ref[0, 0:1, :] = rmax
        gmm_ref[0, 1:2, :] = rmin

    @pl.when(j != 0)
    def _():
        gmm_ref[0, 0:1, :] = jnp.maximum(gmm_ref[0, 0:1, :], rmax)
        gmm_ref[0, 1:2, :] = jnp.minimum(gmm_ref[0, 1:2, :], rmin)

    @pl.when(jnp.logical_and(i == 0, j == 0))
    def _():
        lst_ref[...] = jnp.zeros((2, 1024), jnp.float32)

    lst_ref[0:1, :] += jnp.sum(loc, axis=0, keepdims=True)
    lst_ref[1:2, :] += jnp.sum(loc * loc, axis=0, keepdims=True)


def _zvec_body(gmm_ref, lst_ref, g_ref, b_ref, s0w_ref, z_ref):
    al, be = _affine(lst_ref[...], g_ref[...], b_ref[...], CNT_P)
    gsel = jnp.where(al >= 0, gmm_ref[0, 0:1, :], gmm_ref[0, 1:2, :])
    gv = jnp.maximum(gsel * al + be, 0.0)
    z_ref[0] = _dotb(gv, s0w_ref[:, :1024])


def _y0_body(x1_ref, x2_ref, x3_ref, z_ref, s0w_ref, y_ref, st_ref):
    i = pl.program_id(0)
    j = pl.program_id(1)
    y = (_dotb(x1_ref[0], s0w_ref[:, 1024:1088])
         + _dotb(x2_ref[0], s0w_ref[:, 1088:1152])
         + _dotb(x3_ref[0], s0w_ref[:, 1152:])
         + z_ref[0])
    y_ref[0] = y

    @pl.when(jnp.logical_and(i == 0, j == 0))
    def _():
        st_ref[...] = jnp.zeros((2, 512), jnp.float32)

    st_ref[0:1, :] += jnp.sum(y, axis=0, keepdims=True)
    st_ref[1:2, :] += jnp.sum(y * y, axis=0, keepdims=True)


def _y1_body(y0_ref, st0_ref, g_ref, b_ref, s1w_ref, y_ref, st_ref):
    i = pl.program_id(0)
    j = pl.program_id(1)
    al, be = _affine(st0_ref[...], g_ref[...], b_ref[...], CNT_P)
    a = jnp.maximum(y0_ref[0] * al + be, 0.0)
    y = _dotb(a, s1w_ref[...])
    y_ref[0] = y

    @pl.when(jnp.logical_and(i == 0, j == 0))
    def _():
        st_ref[...] = jnp.zeros((2, 256), jnp.float32)

    st_ref[0:1, :] += jnp.sum(y, axis=0, keepdims=True)
    st_ref[1:2, :] += jnp.sum(y * y, axis=0, keepdims=True)


def _logit_body(y1_ref, st1_ref, g_ref, b_ref, lw_ref, lb_ref, o_ref):
    al, be = _affine(st1_ref[...], g_ref[...], b_ref[...], CNT_P)
    a = jnp.maximum(y1_ref[0] * al + be, 0.0)
    lg = _dotb(a, lw_ref[...]) + lb_ref[...]
    o_ref[0] = lg.T


def _bs(shape, imap):
    return pl.BlockSpec(shape, imap)


def _full(shape):
    return pl.BlockSpec(shape, lambda *_: tuple(0 for _ in shape))


def _pc(body, grid, in_specs, out_specs, out_shape):
    return pl.pallas_call(
        body, grid=grid, in_specs=in_specs, out_specs=out_specs,
        out_shape=out_shape, interpret=_INTERPRET)


_ROWS = B * K * N
_NW = 32
_PW = _ROWS // _NW
_CH = 1024
_NCH = _PW // _CH


@functools.cache
def _sc_gather(d):
    @functools.partial(
        pl.kernel,
        mesh=plsc.VectorSubcoreMesh(core_axis_name="c", subcore_axis_name="s"),
        out_type=jax.ShapeDtypeStruct((_ROWS, d), jnp.float32),
        scratch_types=[
            pltpu.VMEM((8, 128), jnp.int32),
            pltpu.VMEM((_CH, d), jnp.float32),
            pltpu.SemaphoreType.DMA,
        ],
        compiler_params=pltpu.CompilerParams(use_tc_tiling_on_sc=False),
    )
    def gather(table_hbm, idx_hbm, out_hbm, idx_v, rows_v, sem):
        cid = lax.axis_index("c")
        sid = lax.axis_index("s")
        wid = sid * 2 + cid
        base = wid * _PW

        def step(i, carry):
            off = pl.multiple_of(base + i * _CH, _CH)
            pltpu.sync_copy(idx_hbm.at[pl.ds(pl.multiple_of(off // 128, 8), 8)],
                            idx_v)
            cps = [pltpu.async_copy(table_hbm.at[idx_v.at[j]],
                                    rows_v.at[pl.ds(j * 128, 128)], sem)
                   for j in range(8)]
            for cp in cps:
                cp.wait()
            pltpu.sync_copy(rows_v, out_hbm.at[pl.ds(off, _CH)])
            return carry

        lax.fori_loop(0, _NCH, step, 0)

    return gather


def _gather_rows(x_flat, idx2):
    return _sc_gather(x_flat.shape[1])(x_flat, idx2)


def kernel(points, knn_ind,
           ec0_w0, ec0_g0, ec0_b0, ec0_w1, ec0_g1, ec0_b1,
           ec1_w0, ec1_g0, ec1_b0, ec1_w1, ec1_g1, ec1_b1,
           ec2_w0, ec2_g0, ec2_b0, ec2_w1, ec2_g1, ec2_b1,
           local_w, local_g, local_b,
           seg0_w, seg0_g, seg0_b,
           seg1_w, seg1_g, seg1_b,
           logit_w, logit_b):
    r1 = lambda a: a.reshape(1, -1)
    ecw = [(ec0_w0, r1(ec0_g0), r1(ec0_b0), ec0_w1, r1(ec0_g1), r1(ec0_b1)),
           (ec1_w0, r1(ec1_g0), r1(ec1_b0), ec1_w1, r1(ec1_g1), r1(ec1_b1)),
           (ec2_w0, r1(ec2_g0), r1(ec2_b0), ec2_w1, r1(ec2_g1), r1(ec2_b1))]

    x0 = jnp.transpose(points, (0, 2, 1))
    x0 = jnp.pad(x0, ((0, 0), (0, 0), (0, 16 - C_IN)))
    w00 = jnp.pad(ec0_w0[:, :C_IN], ((0, 0), (0, 16 - C_IN)))
    w00 = jnp.concatenate(
        [w00, jnp.pad(ec0_w0[:, C_IN:], ((0, 0), (0, 16 - C_IN)))], axis=1)

    knn_t = jnp.transpose(knn_ind, (0, 2, 1))
    idx_g = (knn_t + (jnp.arange(B, dtype=jnp.int32) * N)[:, None, None])
    idx2 = idx_g.reshape(_ROWS // 128, 128).astype(jnp.int32)

    grid = (B, NT)
    b_nt64 = _bs((1, TN, 64), lambda i, j: (i, j, 0))
    st64 = _bs((2, 64), lambda i, j: (0, 0))

    feats = []
    mx = mn = st2 = None
    for li in range(3):
        _, g0, b0, w1, g1, b1 = ecw[li]
        if li == 0:
            x, cp, w0 = x0, 16, w00
        else:
            gp, bp = ecw[li - 1][4], ecw[li - 1][5]
            x = _pc(
                _xnext_body, grid,
                [b_nt64, b_nt64, st64, _full((1, 64)), _full((1, 64))],
                b_nt64,
                jax.ShapeDtypeStruct((B, N, 64), jnp.float32),
            )(mx, mn, st2, gp, bp)
            feats.append(x)
            cp, w0 = 64, ecw[li][0]
        xg = _gather_rows(x.reshape(B * N, cp), idx2).reshape(B, K, N, cp)
        b_ntc = _bs((1, TN, cp), lambda i, j: (i, j, 0))
        b_xg = _bs((1, K, TN, cp), lambda i, j: (i, 0, j, 0))
        st1 = _pc(
            _stats_body, grid,
            [b_xg, b_ntc, _full((64, 2 * cp))],
            st64,
            jax.ShapeDtypeStruct((2, 64), jnp.float32),
        )(xg, x, w0)
        mx, mn, st2 = _pc(
            _edge_body, grid,
            [b_xg, b_ntc, _full((64, 2 * cp)), st64, _full((1, 64)),
             _full((1, 64)), _full((64, 64))],
            [b_nt64, b_nt64, st64],
            [jax.ShapeDtypeStruct((B, N, 64), jnp.float32),
             jax.ShapeDtypeStruct((B, N, 64), jnp.float32),
             jax.ShapeDtypeStruct((2, 64), jnp.float32)],
        )(xg, x, w0, st1, g0, b0, w1)
    x1, x2 = feats

    x3, gmm, lst = _pc(
        _local_body, grid,
        [b_nt64, b_nt64, st64, _full((1, 64)), _full((1, 64)),
         b_nt64, b_nt64, _full((1024, 192))],
        [b_nt64, _bs((1, 2, 1024), lambda i, j: (i, 0, 0)),
         _bs((2, 1024), lambda i, j: (0, 0))],
        [jax.ShapeDtypeStruct((B, N, 64), jnp.float32),
         jax.ShapeDtypeStruct((B, 2, 1024), jnp.float32),
         jax.ShapeDtypeStruct((2, 1024), jnp.float32)],
    )(mx, mn, st2, ecw[2][4], ecw[2][5], x1, x2, local_w)

    z = _pc(
        _zvec_body, (B,),
        [_bs((1, 2, 1024), lambda i: (i, 0, 0)),
         _bs((2, 1024), lambda i: (0, 0)),
         _bs((1, 1024), lambda i: (0, 0)), _bs((1, 1024), lambda i: (0, 0)),
         _bs((512, 1216), lambda i: (0, 0))],
        _bs((1, 1, 512), lambda i: (i, 0, 0)),
        jax.ShapeDtypeStruct((B, 1, 512), jnp.float32),
    )(gmm, lst, r1(local_g), r1(local_b), seg0_w)

    y0, st0 = _pc(
        _y0_body, grid,
        [b_nt64, b_nt64, b_nt64, _bs((1, 1, 512), lambda i, j: (i, 0, 0)),
         _full((512, 1216))],
        [_bs((1, TN, 512), lambda i, j: (i, j, 0)),
         _bs((2, 512), lambda i, j: (0, 0))],
        [jax.ShapeDtypeStruct((B, N, 512), jnp.float32),
         jax.ShapeDtypeStruct((2, 512), jnp.float32)],
    )(x1, x2, x3, z, seg0_w)

    y1, st1s = _pc(
        _y1_body, grid,
        [_bs((1, TN, 512), lambda i, j: (i, j, 0)),
         _bs((2, 512), lambda i, j: (0, 0)),
         _full((1, 512)), _full((1, 512)), _full((256, 512))],
        [_bs((1, TN, 256), lambda i, j: (i, j, 0)),
         _bs((2, 256), lambda i, j: (0, 0))],
        [jax.ShapeDtypeStruct((B, N, 256), jnp.float32),
         jax.ShapeDtypeStruct((2, 256), jnp.float32)],
    )(y0, st0, r1(seg0_g), r1(seg0_b), seg1_w)

    out = _pc(
        _logit_body, grid,
        [_bs((1, TN, 256), lambda i, j: (i, j, 0)),
         _bs((2, 256), lambda i, j: (0, 0)),
         _full((1, 256)), _full((1, 256)), _full((13, 256)), _full((1, 13))],
        _bs((1, 13, TN), lambda i, j: (i, 0, j)),
        jax.ShapeDtypeStruct((B, 13, N), jnp.float32),
    )(y1, st1s, r1(seg1_g), r1(seg1_b), logit_w, r1(logit_b))

    return out


# speedup vs baseline: 1.0127x; 1.0127x over previous
import functools

import jax
import jax.numpy as jnp
from jax import lax
from jax.experimental import pallas as pl
from jax.experimental.pallas import tpu as pltpu
from jax.experimental.pallas import tpu_sc as plsc

B, C_IN, N, K = 4, 9, 4096, 20
TN = 256
NT = N // TN
EPS = 1e-5
CNT_E = float(B * N * K)
CNT_P = float(B * N)
_INTERPRET = False


def _dotb(x, w):
    return lax.dot_general(x.astype(jnp.bfloat16), w.astype(jnp.bfloat16),
                           (((1,), (1,)), ((), ())),
                           preferred_element_type=jnp.float32)


def _affine(s, g, b, cnt):
    m = s[0:1, :] / cnt
    v = s[1:2, :] / cnt - m * m
    al = g * lax.rsqrt(v + EPS)
    be = b - al * m
    return al, be


def _xnext_body(mx_ref, mn_ref, st_ref, g_ref, b_ref, x_ref):
    al, be = _affine(st_ref[...], g_ref[...], b_ref[...], CNT_E)
    x_ref[0] = jnp.maximum(jnp.where(al >= 0, mx_ref[0], mn_ref[0]) * al + be,
                           0.0)


def _edge_h1(xg_ref, x_ref, w0_ref):
    cp = x_ref.shape[2]
    xc = x_ref[0]
    wc = w0_ref[:, :cp]
    wd = w0_ref[:, cp:]
    hc = _dotb(xc, wc)
    xg2 = xg_ref[0].reshape(K * TN, cp)
    xcr = jnp.broadcast_to(xc[None], (K, TN, cp)).reshape(K * TN, cp)
    hcr = jnp.broadcast_to(hc[None], (K, TN, 64)).reshape(K * TN, 64)
    return hcr + _dotb(xg2 - xcr, wd)


def _stats_body(xg_ref, x_ref, w0_ref, st_ref):
    i = pl.program_id(0)
    j = pl.program_id(1)
    h = _edge_h1(xg_ref, x_ref, w0_ref)

    @pl.when(jnp.logical_and(i == 0, j == 0))
    def _():
        st_ref[...] = jnp.zeros((2, 64), jnp.float32)

    st_ref[0:1, :] += jnp.sum(h, axis=0, keepdims=True)
    st_ref[1:2, :] += jnp.sum(h * h, axis=0, keepdims=True)


def _edge_body(xg_ref, x_ref, w0_ref, st1_ref, g_ref, b_ref, w2_ref,
               mx_ref, mn_ref, st2_ref):
    i = pl.program_id(0)
    j = pl.program_id(1)
    al, be = _affine(st1_ref[...], g_ref[...], b_ref[...], CNT_E)
    h = _edge_h1(xg_ref, x_ref, w0_ref)
    a = jnp.maximum(h * al + be, 0.0)
    h2 = _dotb(a, w2_ref[...]).reshape(K, TN, 64)
    mx_ref[0] = jnp.max(h2, axis=0)
    mn_ref[0] = jnp.min(h2, axis=0)

    @pl.when(jnp.logical_and(i == 0, j == 0))
    def _():
        st2_ref[...] = jnp.zeros((2, 64), jnp.float32)

    h2f = h2.reshape(K * TN, 64)
    st2_ref[0:1, :] += jnp.sum(h2f, axis=0, keepdims=True)
    st2_ref[1:2, :] += jnp.sum(h2f * h2f, axis=0, keepdims=True)


def _local_body(mx_ref, mn_ref, st_ref, g_ref, b_ref, x1_ref, x2_ref, lw_ref,
                x3_ref, gmm_ref, lst_ref):
    i = pl.program_id(0)
    j = pl.program_id(1)
    al, be = _affine(st_ref[...], g_ref[...], b_ref[...], CNT_E)
    x3 = jnp.maximum(jnp.where(al >= 0, mx_ref[0], mn_ref[0]) * al + be, 0.0)
    x3_ref[0] = x3
    loc = (_dotb(x1_ref[0], lw_ref[:, :64])
           + _dotb(x2_ref[0], lw_ref[:, 64:128])
           + _dotb(x3, lw_ref[:, 128:]))
    rmax = jnp.max(loc, axis=0, keepdims=True)
    rmin = jnp.min(loc, axis=0, keepdims=True)

    @pl.when(j == 0)
    def _():
        gmm_ref[0, 0:1, :] = rmax
        gmm_ref[0, 1:2, :] = rmin

    @pl.when(j != 0)
    def _():
        gmm_ref[0, 0:1, :] = jnp.maximum(gmm_ref[0, 0:1, :], rmax)
        gmm_ref[0, 1:2, :] = jnp.minimum(gmm_ref[0, 1:2, :], rmin)

    @pl.when(jnp.logical_and(i == 0, j == 0))
    def _():
        lst_ref[...] = jnp.zeros((2, 1024), jnp.float32)

    lst_ref[0:1, :] += jnp.sum(loc, axis=0, keepdims=True)
    lst_ref[1:2, :] += jnp.sum(loc * loc, axis=0, keepdims=True)


def _zvec_body(gmm_ref, lst_ref, g_ref, b_ref, s0w_ref, z_ref):
    al, be = _affine(lst_ref[...], g_ref[...], b_ref[...], CNT_P)
    gsel = jnp.where(al >= 0, gmm_ref[0, 0:1, :], gmm_ref[0, 1:2, :])
    gv = jnp.maximum(gsel * al + be, 0.0)
    z_ref[0] = _dotb(gv, s0w_ref[:, :1024])


def _y0_body(x1_ref, x2_ref, x3_ref, z_ref, s0w_ref, y_ref, st_ref):
    i = pl.program_id(0)
    j = pl.program_id(1)
    y = (_dotb(x1_ref[0], s0w_ref[:, 1024:1088])
         + _dotb(x2_ref[0], s0w_ref[:, 1088:1152])
         + _dotb(x3_ref[0], s0w_ref[:, 1152:])
         + z_ref[0])
    y_ref[0] = y

    @pl.when(jnp.logical_and(i == 0, j == 0))
    def _():
        st_ref[...] = jnp.zeros((2, 512), jnp.float32)

    st_ref[0:1, :] += jnp.sum(y, axis=0, keepdims=True)
    st_ref[1:2, :] += jnp.sum(y * y, axis=0, keepdims=True)


def _y1_body(y0_ref, st0_ref, g_ref, b_ref, s1w_ref, y_ref, st_ref):
    i = pl.program_id(0)
    j = pl.program_id(1)
    al, be = _affine(st0_ref[...], g_ref[...], b_ref[...], CNT_P)
    a = jnp.maximum(y0_ref[0] * al + be, 0.0)
    y = _dotb(a, s1w_ref[...])
    y_ref[0] = y

    @pl.when(jnp.logical_and(i == 0, j == 0))
    def _():
        st_ref[...] = jnp.zeros((2, 256), jnp.float32)

    st_ref[0:1, :] += jnp.sum(y, axis=0, keepdims=True)
    st_ref[1:2, :] += jnp.sum(y * y, axis=0, keepdims=True)


def _logit_body(y1_ref, st1_ref, g_ref, b_ref, lw_ref, lb_ref, o_ref):
    al, be = _affine(st1_ref[...], g_ref[...], b_ref[...], CNT_P)
    a = jnp.maximum(y1_ref[0] * al + be, 0.0)
    lg = _dotb(a, lw_ref[...]) + lb_ref[...]
    o_ref[0] = lg.T


def _bs(shape, imap):
    return pl.BlockSpec(shape, imap)


def _full(shape):
    return pl.BlockSpec(shape, lambda *_: tuple(0 for _ in shape))


def _pc(body, grid, in_specs, out_specs, out_shape):
    return pl.pallas_call(
        body, grid=grid, in_specs=in_specs, out_specs=out_specs,
        out_shape=out_shape, interpret=_INTERPRET)


_ROWS = B * K * N
_NW = 32
_PW = _ROWS // _NW
_CH = 512
_NCH = _PW // _CH


@functools.cache
def _sc_gather(d):
    @functools.partial(
        pl.kernel,
        mesh=plsc.VectorSubcoreMesh(core_axis_name="c", subcore_axis_name="s"),
        out_type=jax.ShapeDtypeStruct((_ROWS, d), jnp.float32),
        scratch_types=[
            pltpu.VMEM((4, 128), jnp.int32),
            pltpu.VMEM((4, 128), jnp.int32),
            pltpu.VMEM((_CH, d), jnp.float32),
            pltpu.VMEM((_CH, d), jnp.float32),
            pltpu.SemaphoreType.DMA,
            pltpu.SemaphoreType.DMA,
        ],
        compiler_params=pltpu.CompilerParams(use_tc_tiling_on_sc=False),
    )
    def gather(table_hbm, idx_hbm, out_hbm, iv0, iv1, rv0, rv1, sg0, sg1):
        cid = lax.axis_index("c")
        sid = lax.axis_index("s")
        wid = sid * 2 + cid
        base = wid * _NCH
        ivs, rvs, sgs = (iv0, iv1), (rv0, rv1), (sg0, sg1)

        def fire(ci, buf):
            pltpu.sync_copy(idx_hbm.at[ci], ivs[buf])
            for j in range(4):
                pltpu.async_copy(table_hbm.at[ivs[buf].at[j]],
                                 rvs[buf].at[pl.ds(j * 128, 128)], sgs[buf])

        def drain_wb(ci, buf):
            pltpu.make_async_copy(out_hbm.at[pl.ds(0, _CH)], rvs[buf],
                                  sgs[buf]).wait()
            off = pl.multiple_of(ci * _CH, _CH)
            pltpu.sync_copy(rvs[buf], out_hbm.at[pl.ds(off, _CH)])

        fire(base, 0)

        def step(t, carry):
            c0 = base + 2 * t
            fire(c0 + 1, 1)
            drain_wb(c0, 0)

            @pl.when(t + 1 < _NCH // 2)
            def _():
                fire(c0 + 2, 0)

            drain_wb(c0 + 1, 1)
            return carry

        lax.fori_loop(0, _NCH // 2, step, 0)

    return gather


def _gather_rows(x_flat, idx3):
    return _sc_gather(x_flat.shape[1])(x_flat, idx3)


def kernel(points, knn_ind,
           ec0_w0, ec0_g0, ec0_b0, ec0_w1, ec0_g1, ec0_b1,
           ec1_w0, ec1_g0, ec1_b0, ec1_w1, ec1_g1, ec1_b1,
           ec2_w0, ec2_g0, ec2_b0, ec2_w1, ec2_g1, ec2_b1,
           local_w, local_g, local_b,
           seg0_w, seg0_g, seg0_b,
           seg1_w, seg1_g, seg1_b,
           logit_w, logit_b):
    r1 = lambda a: a.reshape(1, -1)
    ecw = [(ec0_w0, r1(ec0_g0), r1(ec0_b0), ec0_w1, r1(ec0_g1), r1(ec0_b1)),
           (ec1_w0, r1(ec1_g0), r1(ec1_b0), ec1_w1, r1(ec1_g1), r1(ec1_b1)),
           (ec2_w0, r1(ec2_g0), r1(ec2_b0), ec2_w1, r1(ec2_g1), r1(ec2_b1))]

    x0 = jnp.transpose(points, (0, 2, 1))
    x0 = jnp.pad(x0, ((0, 0), (0, 0), (0, 16 - C_IN)))
    w00 = jnp.pad(ec0_w0[:, :C_IN], ((0, 0), (0, 16 - C_IN)))
    w00 = jnp.concatenate(
        [w00, jnp.pad(ec0_w0[:, C_IN:], ((0, 0), (0, 16 - C_IN)))], axis=1)

    knn_t = jnp.transpose(knn_ind, (0, 2, 1))
    idx_g = (knn_t + (jnp.arange(B, dtype=jnp.int32) * N)[:, None, None])
    idx3 = idx_g.reshape(_ROWS // _CH, 4, 128).astype(jnp.int32)

    grid = (B, NT)
    b_nt64 = _bs((1, TN, 64), lambda i, j: (i, j, 0))
    st64 = _bs((2, 64), lambda i, j: (0, 0))

    feats = []
    mx = mn = st2 = None
    for li in range(3):
        _, g0, b0, w1, g1, b1 = ecw[li]
        if li == 0:
            x, cp, w0 = x0, 16, w00
        else:
            gp, bp = ecw[li - 1][4], ecw[li - 1][5]
            x = _pc(
                _xnext_body, grid,
                [b_nt64, b_nt64, st64, _full((1, 64)), _full((1, 64))],
                b_nt64,
                jax.ShapeDtypeStruct((B, N, 64), jnp.float32),
            )(mx, mn, st2, gp, bp)
            feats.append(x)
            cp, w0 = 64, ecw[li][0]
        xg = _gather_rows(x.reshape(B * N, cp), idx3).reshape(B, K, N, cp)
        b_ntc = _bs((1, TN, cp), lambda i, j: (i, j, 0))
        b_xg = _bs((1, K, TN, cp), lambda i, j: (i, 0, j, 0))
        st1 = _pc(
            _stats_body, grid,
            [b_xg, b_ntc, _full((64, 2 * cp))],
            st64,
            jax.ShapeDtypeStruct((2, 64), jnp.float32),
        )(xg, x, w0)
        mx, mn, st2 = _pc(
            _edge_body, grid,
            [b_xg, b_ntc, _full((64, 2 * cp)), st64, _full((1, 64)),
             _full((1, 64)), _full((64, 64))],
            [b_nt64, b_nt64, st64],
            [jax.ShapeDtypeStruct((B, N, 64), jnp.float32),
             jax.ShapeDtypeStruct((B, N, 64), jnp.float32),
             jax.ShapeDtypeStruct((2, 64), jnp.float32)],
        )(xg, x, w0, st1, g0, b0, w1)
    x1, x2 = feats

    x3, gmm, lst = _pc(
        _local_body, grid,
        [b_nt64, b_nt64, st64, _full((1, 64)), _full((1, 64)),
         b_nt64, b_nt64, _full((1024, 192))],
        [b_nt64, _bs((1, 2, 1024), lambda i, j: (i, 0, 0)),
         _bs((2, 1024), lambda i, j: (0, 0))],
        [jax.ShapeDtypeStruct((B, N, 64), jnp.float32),
         jax.ShapeDtypeStruct((B, 2, 1024), jnp.float32),
         jax.ShapeDtypeStruct((2, 1024), jnp.float32)],
    )(mx, mn, st2, ecw[2][4], ecw[2][5], x1, x2, local_w)

    z = _pc(
        _zvec_body, (B,),
        [_bs((1, 2, 1024), lambda i: (i, 0, 0)),
         _bs((2, 1024), lambda i: (0, 0)),
         _bs((1, 1024), lambda i: (0, 0)), _bs((1, 1024), lambda i: (0, 0)),
         _bs((512, 1216), lambda i: (0, 0))],
        _bs((1, 1, 512), lambda i: (i, 0, 0)),
        jax.ShapeDtypeStruct((B, 1, 512), jnp.float32),
    )(gmm, lst, r1(local_g), r1(local_b), seg0_w)

    y0, st0 = _pc(
        _y0_body, grid,
        [b_nt64, b_nt64, b_nt64, _bs((1, 1, 512), lambda i, j: (i, 0, 0)),
         _full((512, 1216))],
        [_bs((1, TN, 512), lambda i, j: (i, j, 0)),
         _bs((2, 512), lambda i, j: (0, 0))],
        [jax.ShapeDtypeStruct((B, N, 512), jnp.float32),
         jax.ShapeDtypeStruct((2, 512), jnp.float32)],
    )(x1, x2, x3, z, seg0_w)

    y1, st1s = _pc(
        _y1_body, grid,
        [_bs((1, TN, 512), lambda i, j: (i, j, 0)),
         _bs((2, 512), lambda i, j: (0, 0)),
         _full((1, 512)), _full((1, 512)), _full((256, 512))],
        [_bs((1, TN, 256), lambda i, j: (i, j, 0)),
         _bs((2, 256), lambda i, j: (0, 0))],
        [jax.ShapeDtypeStruct((B, N, 256), jnp.float32),
         jax.ShapeDtypeStruct((2, 256), jnp.float32)],
    )(y0, st0, r1(seg0_g), r1(seg0_b), seg1_w)

    out = _pc(
        _logit_body, grid,
        [_bs((1, TN, 256), lambda i, j: (i, j, 0)),
         _bs((2, 256), lambda i, j: (0, 0)),
         _full((1, 256)), _full((1, 256)), _full((13, 256)), _full((1, 13))],
        _bs((1, 13, TN), lambda i, j: (i, 0, j)),
        jax.ShapeDtypeStruct((B, 13, N), jnp.float32),
    )(y1, st1s, r1(seg1_g), r1(seg1_b), logit_w, r1(logit_b))

    return out


# speedup vs baseline: 1.1332x; 1.1190x over previous
import functools

import jax
import jax.numpy as jnp
from jax import lax
from jax.experimental import pallas as pl
from jax.experimental.pallas import tpu as pltpu
from jax.experimental.pallas import tpu_sc as plsc

B, C_IN, N, K = 4, 9, 4096, 20
TN = 512
NT = N // TN
EPS = 1e-5
CNT_E = float(B * N * K)
CNT_P = float(B * N)
_INTERPRET = False


def _dotb(x, w):
    return lax.dot_general(x.astype(jnp.bfloat16), w.astype(jnp.bfloat16),
                           (((1,), (1,)), ((), ())),
                           preferred_element_type=jnp.float32)


def _affine(s, g, b, cnt):
    m = s[0:1, :] / cnt
    v = s[1:2, :] / cnt - m * m
    al = g * lax.rsqrt(v + EPS)
    be = b - al * m
    return al, be


def _xnext_body(mx_ref, mn_ref, st_ref, g_ref, b_ref, x_ref):
    al, be = _affine(st_ref[...], g_ref[...], b_ref[...], CNT_E)
    x_ref[0] = jnp.maximum(jnp.where(al >= 0, mx_ref[0], mn_ref[0]) * al + be,
                           0.0)


def _edge_h1(xg_ref, x_ref, w0_ref):
    cp = x_ref.shape[2]
    xc = x_ref[0]
    wc = w0_ref[:, :cp]
    wd = w0_ref[:, cp:]
    hc = _dotb(xc, wc)
    xg2 = xg_ref[0].reshape(K * TN, cp)
    xcr = jnp.broadcast_to(xc[None], (K, TN, cp)).reshape(K * TN, cp)
    hcr = jnp.broadcast_to(hc[None], (K, TN, 64)).reshape(K * TN, 64)
    return hcr + _dotb(xg2 - xcr, wd)


def _ec_body(xg_ref, x_ref, w0_ref, g_ref, b_ref, w2_ref,
             mx_ref, mn_ref, st1_ref, st2_ref):
    ph = pl.program_id(0)
    i = pl.program_id(1)
    j = pl.program_id(2)
    first = jnp.logical_and(i == 0, j == 0)
    h = _edge_h1(xg_ref, x_ref, w0_ref)

    @pl.when(ph == 0)
    def _():
        @pl.when(first)
        def _():
            st1_ref[...] = jnp.zeros((2, 64), jnp.float32)

        st1_ref[0:1, :] += jnp.sum(h, axis=0, keepdims=True)
        st1_ref[1:2, :] += jnp.sum(h * h, axis=0, keepdims=True)

    @pl.when(ph == 1)
    def _():
        al, be = _affine(st1_ref[...], g_ref[...], b_ref[...], CNT_E)
        a = jnp.maximum(h * al + be, 0.0)
        h2 = _dotb(a, w2_ref[...]).reshape(K, TN, 64)
        mx_ref[0] = jnp.max(h2, axis=0)
        mn_ref[0] = jnp.min(h2, axis=0)

        @pl.when(first)
        def _():
            st2_ref[...] = jnp.zeros((2, 64), jnp.float32)

        h2f = h2.reshape(K * TN, 64)
        st2_ref[0:1, :] += jnp.sum(h2f, axis=0, keepdims=True)
        st2_ref[1:2, :] += jnp.sum(h2f * h2f, axis=0, keepdims=True)


def _local_body(mx_ref, mn_ref, st_ref, g_ref, b_ref, x1_ref, x2_ref, lw_ref,
                x3_ref, gmm_ref, lst_ref):
    i = pl.program_id(0)
    j = pl.program_id(1)
    al, be = _affine(st_ref[...], g_ref[...], b_ref[...], CNT_E)
    x3 = jnp.maximum(jnp.where(al >= 0, mx_ref[0], mn_ref[0]) * al + be, 0.0)
    x3_ref[0] = x3
    loc = (_dotb(x1_ref[0], lw_ref[:, :64])
           + _dotb(x2_ref[0], lw_ref[:, 64:128])
           + _dotb(x3, lw_ref[:, 128:]))
    rmax = jnp.max(loc, axis=0, keepdims=True)
    rmin = jnp.min(loc, axis=0, keepdims=True)

    @pl.when(j == 0)
    def _():
        gmm_ref[0, 0:1, :] = rmax
        gmm_ref[0, 1:2, :] = rmin

    @pl.when(j != 0)
    def _():
        gmm_ref[0, 0:1, :] = jnp.maximum(gmm_ref[0, 0:1, :], rmax)
        gmm_ref[0, 1:2, :] = jnp.minimum(gmm_ref[0, 1:2, :], rmin)

    @pl.when(jnp.logical_and(i == 0, j == 0))
    def _():
        lst_ref[...] = jnp.zeros((2, 1024), jnp.float32)

    lst_ref[0:1, :] += jnp.sum(loc, axis=0, keepdims=True)
    lst_ref[1:2, :] += jnp.sum(loc * loc, axis=0, keepdims=True)


def _zvec_body(gmm_ref, lst_ref, g_ref, b_ref, s0w_ref, z_ref):
    al, be = _affine(lst_ref[...], g_ref[...], b_ref[...], CNT_P)
    gsel = jnp.where(al >= 0, gmm_ref[0, 0:1, :], gmm_ref[0, 1:2, :])
    gv = jnp.maximum(gsel * al + be, 0.0)
    z_ref[0] = _dotb(gv, s0w_ref[:, :1024])


def _y0_body(x1_ref, x2_ref, x3_ref, z_ref, s0w_ref, y_ref, st_ref):
    i = pl.program_id(0)
    j = pl.program_id(1)
    y = (_dotb(x1_ref[0], s0w_ref[:, 1024:1088])
         + _dotb(x2_ref[0], s0w_ref[:, 1088:1152])
         + _dotb(x3_ref[0], s0w_ref[:, 1152:])
         + z_ref[0])
    y_ref[0] = y

    @pl.when(jnp.logical_and(i == 0, j == 0))
    def _():
        st_ref[...] = jnp.zeros((2, 512), jnp.float32)

    st_ref[0:1, :] += jnp.sum(y, axis=0, keepdims=True)
    st_ref[1:2, :] += jnp.sum(y * y, axis=0, keepdims=True)


def _y1_body(y0_ref, st0_ref, g_ref, b_ref, s1w_ref, y_ref, st_ref):
    i = pl.program_id(0)
    j = pl.program_id(1)
    al, be = _affine(st0_ref[...], g_ref[...], b_ref[...], CNT_P)
    a = jnp.maximum(y0_ref[0] * al + be, 0.0)
    y = _dotb(a, s1w_ref[...])
    y_ref[0] = y

    @pl.when(jnp.logical_and(i == 0, j == 0))
    def _():
        st_ref[...] = jnp.zeros((2, 256), jnp.float32)

    st_ref[0:1, :] += jnp.sum(y, axis=0, keepdims=True)
    st_ref[1:2, :] += jnp.sum(y * y, axis=0, keepdims=True)


def _logit_body(y1_ref, st1_ref, g_ref, b_ref, lw_ref, lb_ref, o_ref):
    al, be = _affine(st1_ref[...], g_ref[...], b_ref[...], CNT_P)
    a = jnp.maximum(y1_ref[0] * al + be, 0.0)
    lg = _dotb(a, lw_ref[...]) + lb_ref[...]
    o_ref[0] = lg.T


def _bs(shape, imap):
    return pl.BlockSpec(shape, imap)


def _full(shape):
    return pl.BlockSpec(shape, lambda *_: tuple(0 for _ in shape))


def _pc(body, grid, in_specs, out_specs, out_shape):
    return pl.pallas_call(
        body, grid=grid, in_specs=in_specs, out_specs=out_specs,
        out_shape=out_shape, interpret=_INTERPRET)


_ROWS = B * K * N
_NW = 32
_PW = _ROWS // _NW
_CH = 512
_NCH = _PW // _CH


@functools.cache
def _sc_gather(d):
    @functools.partial(
        pl.kernel,
        mesh=plsc.VectorSubcoreMesh(core_axis_name="c", subcore_axis_name="s"),
        out_type=jax.ShapeDtypeStruct((_ROWS, d), jnp.float32),
        scratch_types=[
            pltpu.VMEM((4, 128), jnp.int32),
            pltpu.VMEM((4, 128), jnp.int32),
            pltpu.VMEM((_CH, d), jnp.float32),
            pltpu.VMEM((_CH, d), jnp.float32),
            pltpu.SemaphoreType.DMA,
            pltpu.SemaphoreType.DMA,
        ],
        compiler_params=pltpu.CompilerParams(use_tc_tiling_on_sc=False),
    )
    def gather(table_hbm, idx_hbm, out_hbm, iv0, iv1, rv0, rv1, sg0, sg1):
        cid = lax.axis_index("c")
        sid = lax.axis_index("s")
        wid = sid * 2 + cid
        base = wid * _NCH
        ivs, rvs, sgs = (iv0, iv1), (rv0, rv1), (sg0, sg1)

        def fire(ci, buf):
            pltpu.sync_copy(idx_hbm.at[ci], ivs[buf])
            for j in range(4):
                pltpu.async_copy(table_hbm.at[ivs[buf].at[j]],
                                 rvs[buf].at[pl.ds(j * 128, 128)], sgs[buf])

        def drain_wb(ci, buf):
            pltpu.make_async_copy(out_hbm.at[pl.ds(0, _CH)], rvs[buf],
                                  sgs[buf]).wait()
            off = pl.multiple_of(ci * _CH, _CH)
            pltpu.sync_copy(rvs[buf], out_hbm.at[pl.ds(off, _CH)])

        fire(base, 0)

        def step(t, carry):
            c0 = base + 2 * t
            fire(c0 + 1, 1)
            drain_wb(c0, 0)

            @pl.when(t + 1 < _NCH // 2)
            def _():
                fire(c0 + 2, 0)

            drain_wb(c0 + 1, 1)
            return carry

        lax.fori_loop(0, _NCH // 2, step, 0)

    return gather


def _gather_rows(x_flat, idx3):
    return _sc_gather(x_flat.shape[1])(x_flat, idx3)


def kernel(points, knn_ind,
           ec0_w0, ec0_g0, ec0_b0, ec0_w1, ec0_g1, ec0_b1,
           ec1_w0, ec1_g0, ec1_b0, ec1_w1, ec1_g1, ec1_b1,
           ec2_w0, ec2_g0, ec2_b0, ec2_w1, ec2_g1, ec2_b1,
           local_w, local_g, local_b,
           seg0_w, seg0_g, seg0_b,
           seg1_w, seg1_g, seg1_b,
           logit_w, logit_b):
    r1 = lambda a: a.reshape(1, -1)
    ecw = [(ec0_w0, r1(ec0_g0), r1(ec0_b0), ec0_w1, r1(ec0_g1), r1(ec0_b1)),
           (ec1_w0, r1(ec1_g0), r1(ec1_b0), ec1_w1, r1(ec1_g1), r1(ec1_b1)),
           (ec2_w0, r1(ec2_g0), r1(ec2_b0), ec2_w1, r1(ec2_g1), r1(ec2_b1))]

    x0 = jnp.transpose(points, (0, 2, 1))
    x0 = jnp.pad(x0, ((0, 0), (0, 0), (0, 16 - C_IN)))
    w00 = jnp.pad(ec0_w0[:, :C_IN], ((0, 0), (0, 16 - C_IN)))
    w00 = jnp.concatenate(
        [w00, jnp.pad(ec0_w0[:, C_IN:], ((0, 0), (0, 16 - C_IN)))], axis=1)

    knn_t = jnp.transpose(knn_ind, (0, 2, 1))
    idx_g = (knn_t + (jnp.arange(B, dtype=jnp.int32) * N)[:, None, None])
    idx3 = idx_g.reshape(_ROWS // _CH, 4, 128).astype(jnp.int32)

    grid = (B, NT)
    b_nt64 = _bs((1, TN, 64), lambda i, j: (i, j, 0))
    st64 = _bs((2, 64), lambda i, j: (0, 0))

    feats = []
    mx = mn = st2 = None
    for li in range(3):
        _, g0, b0, w1, g1, b1 = ecw[li]
        if li == 0:
            x, cp, w0 = x0, 16, w00
        else:
            gp, bp = ecw[li - 1][4], ecw[li - 1][5]
            x = _pc(
                _xnext_body, grid,
                [b_nt64, b_nt64, st64, _full((1, 64)), _full((1, 64))],
                b_nt64,
                jax.ShapeDtypeStruct((B, N, 64), jnp.float32),
            )(mx, mn, st2, gp, bp)
            feats.append(x)
            cp, w0 = 64, ecw[li][0]
        xg = _gather_rows(x.reshape(B * N, cp), idx3).reshape(B, K, N, cp)
        b_ntc3 = _bs((1, TN, cp), lambda p, i, j: (i, j, 0))
        b_xg3 = _bs((1, K, TN, cp), lambda p, i, j: (i, 0, j, 0))
        b_nt64_3 = _bs((1, TN, 64), lambda p, i, j: (i, j, 0))
        st64_3 = _bs((2, 64), lambda p, i, j: (0, 0))
        f3 = lambda shape: pl.BlockSpec(shape, lambda p, i, j: (0,) * len(shape))
        mx, mn, st1, st2 = _pc(
            _ec_body, (2, B, NT),
            [b_xg3, b_ntc3, f3((64, 2 * cp)), f3((1, 64)), f3((1, 64)),
             f3((64, 64))],
            [b_nt64_3, b_nt64_3, st64_3, st64_3],
            [jax.ShapeDtypeStruct((B, N, 64), jnp.float32),
             jax.ShapeDtypeStruct((B, N, 64), jnp.float32),
             jax.ShapeDtypeStruct((2, 64), jnp.float32),
             jax.ShapeDtypeStruct((2, 64), jnp.float32)],
        )(xg, x, w0, g0, b0, w1)
    x1, x2 = feats

    x3, gmm, lst = _pc(
        _local_body, grid,
        [b_nt64, b_nt64, st64, _full((1, 64)), _full((1, 64)),
         b_nt64, b_nt64, _full((1024, 192))],
        [b_nt64, _bs((1, 2, 1024), lambda i, j: (i, 0, 0)),
         _bs((2, 1024), lambda i, j: (0, 0))],
        [jax.ShapeDtypeStruct((B, N, 64), jnp.float32),
         jax.ShapeDtypeStruct((B, 2, 1024), jnp.float32),
         jax.ShapeDtypeStruct((2, 1024), jnp.float32)],
    )(mx, mn, st2, ecw[2][4], ecw[2][5], x1, x2, local_w)

    z = _pc(
        _zvec_body, (B,),
        [_bs((1, 2, 1024), lambda i: (i, 0, 0)),
         _bs((2, 1024), lambda i: (0, 0)),
         _bs((1, 1024), lambda i: (0, 0)), _bs((1, 1024), lambda i: (0, 0)),
         _bs((512, 1216), lambda i: (0, 0))],
        _bs((1, 1, 512), lambda i: (i, 0, 0)),
        jax.ShapeDtypeStruct((B, 1, 512), jnp.float32),
    )(gmm, lst, r1(local_g), r1(local_b), seg0_w)

    y0, st0 = _pc(
        _y0_body, grid,
        [b_nt64, b_nt64, b_nt64, _bs((1, 1, 512), lambda i, j: (i, 0, 0)),
         _full((512, 1216))],
        [_bs((1, TN, 512), lambda i, j: (i, j, 0)),
         _bs((2, 512), lambda i, j: (0, 0))],
        [jax.ShapeDtypeStruct((B, N, 512), jnp.float32),
         jax.ShapeDtypeStruct((2, 512), jnp.float32)],
    )(x1, x2, x3, z, seg0_w)

    y1, st1s = _pc(
        _y1_body, grid,
        [_bs((1, TN, 512), lambda i, j: (i, j, 0)),
         _bs((2, 512), lambda i, j: (0, 0)),
         _full((1, 512)), _full((1, 512)), _full((256, 512))],
        [_bs((1, TN, 256), lambda i, j: (i, j, 0)),
         _bs((2, 256), lambda i, j: (0, 0))],
        [jax.ShapeDtypeStruct((B, N, 256), jnp.float32),
         jax.ShapeDtypeStruct((2, 256), jnp.float32)],
    )(y0, st0, r1(seg0_g), r1(seg0_b), seg1_w)

    out = _pc(
        _logit_body, grid,
        [_bs((1, TN, 256), lambda i, j: (i, j, 0)),
         _bs((2, 256), lambda i, j: (0, 0)),
         _full((1, 256)), _full((1, 256)), _full((13, 256)), _full((1, 13))],
        _bs((1, 13, TN), lambda i, j: (i, 0, j)),
        jax.ShapeDtypeStruct((B, 13, N), jnp.float32),
    )(y1, st1s, r1(seg1_g), r1(seg1_b), logit_w, r1(logit_b))

    return out
